# baseline (speedup 1.0000x reference)
import functools

import jax
import jax.numpy as jnp
from jax.experimental import pallas as pl

S_DIM = 48
V_DIM = 12
H_DIM = 96
N_LAYERS = 3


def _silu(x):
    return x * jax.nn.sigmoid(x)


def _egnn_kernel(pos_ref, h0_ref,
                 A_ref, Bm_ref, wd_ref, wm_ref, be1_ref,
                 We2_ref, be2_ref, Wx_ref, bx_ref,
                 Wh1h_ref, Wh1a_ref, Wh1v_ref, bh1_ref,
                 Wh2_ref, bh2_ref, Ws_ref, bs_ref,
                 out_ref):
    N = pos_ref.shape[1]
    pos = pos_ref[0]
    centroid = jnp.mean(pos, axis=0, keepdims=True)
    rel = pos[:, None, :] - pos[None, :, :]
    dist2 = jnp.sum(rel * rel, axis=-1)
    mid = 0.5 * (pos[:, None, :] + pos[None, :, :])
    midfeat = jnp.sum((mid - centroid[None, :, :]) ** 2, axis=-1)
    midfeat_diag = jnp.sum((pos - centroid) ** 2, axis=-1)
    Srel = N * (pos - centroid)
    inv = 1.0 / (N - 1)

    h = jnp.broadcast_to(h0_ref[0], (N, S_DIM))
    v0 = jnp.zeros((N, V_DIM), jnp.float32)
    v1 = jnp.zeros((N, V_DIM), jnp.float32)
    v2 = jnp.zeros((N, V_DIM), jnp.float32)

    for l in range(N_LAYERS):
        A = A_ref[l]
        Bm = Bm_ref[l]
        wd = wd_ref[l]
        wm = wm_ref[l]
        be1 = be1_ref[l]
        We2 = We2_ref[l]
        be2 = be2_ref[l]
        Wx = Wx_ref[l]
        bx = bx_ref[l]

        hA = jnp.dot(h, A, preferred_element_type=jnp.float32)
        hB = jnp.dot(h, Bm, preferred_element_type=jnp.float32)
        pre = (hA[:, None, :] + hB[None, :, :]
               + dist2[:, :, None] * wd[None, :, :]
               + midfeat[:, :, None] * wm[None, :, :]
               + be1[None, :, :])
        m1 = _silu(pre)
        m = _silu(jnp.dot(m1.reshape(N * N, H_DIM), We2,
                          preferred_element_type=jnp.float32)
                  + be2).reshape(N, N, H_DIM)

        pre_d = hA + hB + midfeat_diag[:, None] * wm + be1
        m_d = _silu(jnp.dot(_silu(pre_d), We2,
                            preferred_element_type=jnp.float32) + be2)

        agg = (jnp.sum(m, axis=1) - m_d) * inv

        dvs = []
        for c in range(3):
            G = jnp.sum(rel[:, :, c][:, :, None] * m, axis=1)
            dv_c = (jnp.dot(G, Wx, preferred_element_type=jnp.float32)
                    + bx * Srel[:, c][:, None]) * inv
            dvs.append(dv_c)
        v0 = v0 + dvs[0]
        v1 = v1 + dvs[1]
        v2 = v2 + dvs[2]
        vnorm = v0 * v0 + v1 * v1 + v2 * v2

        u_pre = (jnp.dot(h, Wh1h_ref[l], preferred_element_type=jnp.float32)
                 + jnp.dot(agg, Wh1a_ref[l], preferred_element_type=jnp.float32)
                 + jnp.dot(vnorm, Wh1v_ref[l], preferred_element_type=jnp.float32)
                 + bh1_ref[l])
        u = (jnp.dot(_silu(u_pre), Wh2_ref[l],
                     preferred_element_type=jnp.float32) + bh2_ref[l])
        h = h + u

    out = jnp.dot(h, Ws_ref[0], preferred_element_type=jnp.float32) + bs_ref[0]
    out_ref[0, 0] = out[:, 0]


@functools.partial(jax.jit, static_argnames=("interpret",))
def _run(positions, flat_weights, interpret=False):
    B, N, _ = positions.shape
    (h0, A, Bm, wd, wm, be1, We2, be2, Wx, bx,
     Wh1h, Wh1a, Wh1v, bh1, Wh2, bh2, Ws, bs) = flat_weights

    def full(x):
        return pl.BlockSpec(x.shape, lambda b: (0,) * x.ndim)

    in_specs = [pl.BlockSpec((1, N, 3), lambda b: (b, 0, 0))]
    in_specs += [full(x) for x in flat_weights]

    out = pl.pallas_call(
        _egnn_kernel,
        grid=(B,),
        in_specs=in_specs,
        out_specs=pl.BlockSpec((1, 1, N), lambda b: (b, 0, 0)),
        out_shape=jax.ShapeDtypeStruct((B, 1, N), jnp.float32),
        interpret=interpret,
    )(positions, *flat_weights)
    return out.reshape(B, N)


def _flatten_params(params):
    ls = params["layers"]

    def stack(f):
        return jnp.stack([f(l) for l in ls])

    h0 = params["h0"].reshape(1, S_DIM)
    A = stack(lambda l: l["We1"][:S_DIM])
    Bm = stack(lambda l: l["We1"][S_DIM:2 * S_DIM])
    wd = stack(lambda l: l["We1"][2 * S_DIM:2 * S_DIM + 1])
    wm = stack(lambda l: l["We1"][2 * S_DIM + 1:2 * S_DIM + 2])
    be1 = stack(lambda l: l["be1"].reshape(1, H_DIM))
    We2 = stack(lambda l: l["We2"])
    be2 = stack(lambda l: l["be2"].reshape(1, H_DIM))
    Wx = stack(lambda l: l["Wx"])
    bx = stack(lambda l: l["bx"].reshape(1, V_DIM))
    Wh1h = stack(lambda l: l["Wh1"][:S_DIM])
    Wh1a = stack(lambda l: l["Wh1"][S_DIM:S_DIM + H_DIM])
    Wh1v = stack(lambda l: l["Wh1"][S_DIM + H_DIM:])
    bh1 = stack(lambda l: l["bh1"].reshape(1, H_DIM))
    Wh2 = stack(lambda l: l["Wh2"])
    bh2 = stack(lambda l: l["bh2"].reshape(1, S_DIM))
    Ws = params["Ws"].reshape(1, S_DIM, 1)
    bs = params["bs"].reshape(1, 1)
    return (h0, A, Bm, wd, wm, be1, We2, be2, Wx, bx,
            Wh1h, Wh1a, Wh1v, bh1, Wh2, bh2, Ws, bs)


def kernel(positions, params):
    return _run(positions, _flatten_params(params))


# speedup vs baseline: 1.0006x; 1.0006x over previous
import functools

import jax
import jax.numpy as jnp
from jax.experimental import pallas as pl
from jax.experimental.pallas import tpu as pltpu

S_DIM = 48
V_DIM = 12
H_DIM = 96
N_LAYERS = 3


def _silu(x):
    return x * jax.nn.sigmoid(x)


def _egnn_kernel(pos_ref, h0_ref,
                 A_ref, Bm_ref, wd_ref, wm_ref, be1_ref,
                 We2_ref, be2_ref, Wx_ref, bx_ref,
                 Wh1h_ref, Wh1a_ref, Wh1v_ref, bh1_ref,
                 Wh2_ref, bh2_ref, Ws_ref, bs_ref,
                 out_ref):
    N = pos_ref.shape[1]
    pos = pos_ref[0]
    centroid = jnp.mean(pos, axis=0, keepdims=True)
    rel = pos[:, None, :] - pos[None, :, :]
    dist2 = jnp.sum(rel * rel, axis=-1)
    mid = 0.5 * (pos[:, None, :] + pos[None, :, :])
    midfeat = jnp.sum((mid - centroid[None, :, :]) ** 2, axis=-1)
    midfeat_diag = jnp.sum((pos - centroid) ** 2, axis=-1)
    Srel = N * (pos - centroid)
    inv = 1.0 / (N - 1)

    h = jnp.broadcast_to(h0_ref[0], (N, S_DIM))
    v0 = jnp.zeros((N, V_DIM), jnp.float32)
    v1 = jnp.zeros((N, V_DIM), jnp.float32)
    v2 = jnp.zeros((N, V_DIM), jnp.float32)

    for l in range(N_LAYERS):
        A = A_ref[l]
        Bm = Bm_ref[l]
        wd = wd_ref[l]
        wm = wm_ref[l]
        be1 = be1_ref[l]
        We2 = We2_ref[l]
        be2 = be2_ref[l]
        Wx = Wx_ref[l]
        bx = bx_ref[l]

        hA = jnp.dot(h, A, preferred_element_type=jnp.float32)
        hB = jnp.dot(h, Bm, preferred_element_type=jnp.float32)
        pre = (hA[:, None, :] + hB[None, :, :]
               + dist2[:, :, None] * wd[None, :, :]
               + midfeat[:, :, None] * wm[None, :, :]
               + be1[None, :, :])
        m1 = _silu(pre)
        m = _silu(jnp.dot(m1.reshape(N * N, H_DIM), We2,
                          preferred_element_type=jnp.float32)
                  + be2).reshape(N, N, H_DIM)

        pre_d = hA + hB + midfeat_diag[:, None] * wm + be1
        m_d = _silu(jnp.dot(_silu(pre_d), We2,
                            preferred_element_type=jnp.float32) + be2)

        agg = (jnp.sum(m, axis=1) - m_d) * inv

        dvs = []
        for c in range(3):
            G = jnp.sum(rel[:, :, c][:, :, None] * m, axis=1)
            dv_c = (jnp.dot(G, Wx, preferred_element_type=jnp.float32)
                    + bx * Srel[:, c][:, None]) * inv
            dvs.append(dv_c)
        v0 = v0 + dvs[0]
        v1 = v1 + dvs[1]
        v2 = v2 + dvs[2]
        vnorm = v0 * v0 + v1 * v1 + v2 * v2

        u_pre = (jnp.dot(h, Wh1h_ref[l], preferred_element_type=jnp.float32)
                 + jnp.dot(agg, Wh1a_ref[l], preferred_element_type=jnp.float32)
                 + jnp.dot(vnorm, Wh1v_ref[l], preferred_element_type=jnp.float32)
                 + bh1_ref[l])
        u = (jnp.dot(_silu(u_pre), Wh2_ref[l],
                     preferred_element_type=jnp.float32) + bh2_ref[l])
        h = h + u

    out = jnp.dot(h, Ws_ref[0], preferred_element_type=jnp.float32) + bs_ref[0]
    out_ref[0, 0] = out[:, 0]


@functools.partial(jax.jit, static_argnames=("interpret",))
def _run(positions, flat_weights, interpret=False):
    B, N, _ = positions.shape
    (h0, A, Bm, wd, wm, be1, We2, be2, Wx, bx,
     Wh1h, Wh1a, Wh1v, bh1, Wh2, bh2, Ws, bs) = flat_weights

    def full(x):
        return pl.BlockSpec(x.shape, lambda b: (0,) * x.ndim)

    in_specs = [pl.BlockSpec((1, N, 3), lambda b: (b, 0, 0))]
    in_specs += [full(x) for x in flat_weights]

    out = pl.pallas_call(
        _egnn_kernel,
        grid=(B,),
        in_specs=in_specs,
        out_specs=pl.BlockSpec((1, 1, N), lambda b: (b, 0, 0)),
        out_shape=jax.ShapeDtypeStruct((B, 1, N), jnp.float32),
        compiler_params=pltpu.CompilerParams(
            dimension_semantics=("parallel",)),
        interpret=interpret,
    )(positions, *flat_weights)
    return out.reshape(B, N)


def _flatten_params(params):
    ls = params["layers"]

    def stack(f):
        return jnp.stack([f(l) for l in ls])

    h0 = params["h0"].reshape(1, S_DIM)
    A = stack(lambda l: l["We1"][:S_DIM])
    Bm = stack(lambda l: l["We1"][S_DIM:2 * S_DIM])
    wd = stack(lambda l: l["We1"][2 * S_DIM:2 * S_DIM + 1])
    wm = stack(lambda l: l["We1"][2 * S_DIM + 1:2 * S_DIM + 2])
    be1 = stack(lambda l: l["be1"].reshape(1, H_DIM))
    We2 = stack(lambda l: l["We2"])
    be2 = stack(lambda l: l["be2"].reshape(1, H_DIM))
    Wx = stack(lambda l: l["Wx"])
    bx = stack(lambda l: l["bx"].reshape(1, V_DIM))
    Wh1h = stack(lambda l: l["Wh1"][:S_DIM])
    Wh1a = stack(lambda l: l["Wh1"][S_DIM:S_DIM + H_DIM])
    Wh1v = stack(lambda l: l["Wh1"][S_DIM + H_DIM:])
    bh1 = stack(lambda l: l["bh1"].reshape(1, H_DIM))
    Wh2 = stack(lambda l: l["Wh2"])
    bh2 = stack(lambda l: l["bh2"].reshape(1, S_DIM))
    Ws = params["Ws"].reshape(1, S_DIM, 1)
    bs = params["bs"].reshape(1, 1)
    return (h0, A, Bm, wd, wm, be1, We2, be2, Wx, bx,
            Wh1h, Wh1a, Wh1v, bh1, Wh2, bh2, Ws, bs)


def kernel(positions, params):
    return _run(positions, _flatten_params(params))


# speedup vs baseline: 1.0648x; 1.0641x over previous
import functools

import jax
import jax.numpy as jnp
from jax.experimental import pallas as pl
from jax.experimental.pallas import tpu as pltpu

S_DIM = 48
V_DIM = 12
H_DIM = 96
N_LAYERS = 3


def _silu(x):
    return x * jax.nn.sigmoid(x)


def _egnn_kernel(pos_ref, h0_ref,
                 W3_ref,
                 A_ref, Bm_ref,
                 We2_ref, be2_ref, Wx_ref, bx_ref,
                 Wh1h_ref, Wh1a_ref, Wh1v_ref, bh1_ref,
                 Wh2_ref, bh2_ref, Ws_ref, bs_ref,
                 out_ref):
    N = pos_ref.shape[1]
    pos = pos_ref[0]
    centroid = jnp.mean(pos, axis=0, keepdims=True)
    rel = pos[:, None, :] - pos[None, :, :]
    dist2 = jnp.sum(rel * rel, axis=-1)
    mid = 0.5 * (pos[:, None, :] + pos[None, :, :])
    midfeat = jnp.sum((mid - centroid[None, :, :]) ** 2, axis=-1)
    midfeat_diag = jnp.sum((pos - centroid) ** 2, axis=-1)
    ef = jnp.concatenate([dist2[:, :, None], midfeat[:, :, None],
                          jnp.ones((N, N, 1), jnp.float32)],
                         axis=-1).reshape(N * N, 3)
    efd = jnp.concatenate([jnp.zeros((N, 1), jnp.float32),
                           midfeat_diag[:, None],
                           jnp.ones((N, 1), jnp.float32)], axis=-1)
    Srel = N * (pos - centroid)
    inv = 1.0 / (N - 1)

    h = jnp.broadcast_to(h0_ref[0], (N, S_DIM))
    v0 = jnp.zeros((N, V_DIM), jnp.float32)
    v1 = jnp.zeros((N, V_DIM), jnp.float32)
    v2 = jnp.zeros((N, V_DIM), jnp.float32)

    for l in range(N_LAYERS):
        A = A_ref[l]
        Bm = Bm_ref[l]
        W3 = W3_ref[l]
        We2 = We2_ref[l]
        be2 = be2_ref[l]
        Wx = Wx_ref[l]
        bx = bx_ref[l]

        hA = jnp.dot(h, A, preferred_element_type=jnp.float32)
        hB = jnp.dot(h, Bm, preferred_element_type=jnp.float32)
        rank2 = jnp.dot(ef, W3,
                        preferred_element_type=jnp.float32).reshape(N, N, H_DIM)
        pre = rank2 + hA[:, None, :] + hB[None, :, :]
        m1 = _silu(pre)
        m = _silu(jnp.dot(m1.reshape(N * N, H_DIM), We2,
                          preferred_element_type=jnp.float32)
                  + be2).reshape(N, N, H_DIM)

        pre_d = (hA + hB
                 + jnp.dot(efd, W3, preferred_element_type=jnp.float32))
        m_d = _silu(jnp.dot(_silu(pre_d), We2,
                            preferred_element_type=jnp.float32) + be2)

        agg = (jnp.sum(m, axis=1) - m_d) * inv

        dvs = []
        for c in range(3):
            G = jnp.sum(rel[:, :, c][:, :, None] * m, axis=1)
            dv_c = (jnp.dot(G, Wx, preferred_element_type=jnp.float32)
                    + bx * Srel[:, c][:, None]) * inv
            dvs.append(dv_c)
        v0 = v0 + dvs[0]
        v1 = v1 + dvs[1]
        v2 = v2 + dvs[2]
        vnorm = v0 * v0 + v1 * v1 + v2 * v2

        u_pre = (jnp.dot(h, Wh1h_ref[l], preferred_element_type=jnp.float32)
                 + jnp.dot(agg, Wh1a_ref[l], preferred_element_type=jnp.float32)
                 + jnp.dot(vnorm, Wh1v_ref[l], preferred_element_type=jnp.float32)
                 + bh1_ref[l])
        u = (jnp.dot(_silu(u_pre), Wh2_ref[l],
                     preferred_element_type=jnp.float32) + bh2_ref[l])
        h = h + u

    out = jnp.dot(h, Ws_ref[0], preferred_element_type=jnp.float32) + bs_ref[0]
    out_ref[0, 0] = out[:, 0]


@functools.partial(jax.jit, static_argnames=("interpret",))
def _run(positions, flat_weights, interpret=False):
    B, N, _ = positions.shape

    def full(x):
        return pl.BlockSpec(x.shape, lambda b: (0,) * x.ndim)

    in_specs = [pl.BlockSpec((1, N, 3), lambda b: (b, 0, 0))]
    in_specs += [full(x) for x in flat_weights]

    out = pl.pallas_call(
        _egnn_kernel,
        grid=(B,),
        in_specs=in_specs,
        out_specs=pl.BlockSpec((1, 1, N), lambda b: (b, 0, 0)),
        out_shape=jax.ShapeDtypeStruct((B, 1, N), jnp.float32),
        compiler_params=pltpu.CompilerParams(
            dimension_semantics=("parallel",)),
        interpret=interpret,
    )(positions, *flat_weights)
    return out.reshape(B, N)


def _flatten_params(params):
    ls = params["layers"]

    def stack(f):
        return jnp.stack([f(l) for l in ls])

    h0 = params["h0"].reshape(1, S_DIM)
    A = stack(lambda l: l["We1"][:S_DIM])
    Bm = stack(lambda l: l["We1"][S_DIM:2 * S_DIM])
    W3 = stack(lambda l: jnp.concatenate(
        [l["We1"][2 * S_DIM:2 * S_DIM + 2], l["be1"].reshape(1, H_DIM)],
        axis=0))
    We2 = stack(lambda l: l["We2"])
    be2 = stack(lambda l: l["be2"].reshape(1, H_DIM))
    Wx = stack(lambda l: l["Wx"])
    bx = stack(lambda l: l["bx"].reshape(1, V_DIM))
    Wh1h = stack(lambda l: l["Wh1"][:S_DIM])
    Wh1a = stack(lambda l: l["Wh1"][S_DIM:S_DIM + H_DIM])
    Wh1v = stack(lambda l: l["Wh1"][S_DIM + H_DIM:])
    bh1 = stack(lambda l: l["bh1"].reshape(1, H_DIM))
    Wh2 = stack(lambda l: l["Wh2"])
    bh2 = stack(lambda l: l["bh2"].reshape(1, S_DIM))
    Ws = params["Ws"].reshape(1, S_DIM, 1)
    bs = params["bs"].reshape(1, 1)
    return (h0, W3, A, Bm, We2, be2, Wx, bx,
            Wh1h, Wh1a, Wh1v, bh1, Wh2, bh2, Ws, bs)


def kernel(positions, params):
    return _run(positions, _flatten_params(params))
